# baseline (speedup 1.0000x reference)
import functools

import jax
import jax.numpy as jnp
from jax import lax
from jax.experimental import pallas as pl
from jax.experimental.pallas import tpu as pltpu, tpu_sc as plsc

N_FIELDS = 26
VOCAB = 100000
EMBED = 32
BATCH = 16384

_INFO = plsc.get_sparse_core_info()
_NC, _NS = _INFO.num_cores, _INFO.num_subcores
_NW = _NC * _NS
_BPW = BATCH // _NW
_NG = 4
_GSZ = _BPW // _NG


def _body(ids_hbm, tab_hbm, out_hbm, idx_v, rows0, rows1, gsem, wsem0, wsem1):
    wid = lax.axis_index("s") * _NC + lax.axis_index("c")
    base = wid * _BPW

    pltpu.sync_copy(ids_hbm.at[:, wid], idx_v)

    def add_off(t, _):
        f = t // _NG
        g = t - f * _NG
        for u in range(_GSZ // 16):
            s16 = pl.ds(u * 16, 16)
            idx_v[f, g, s16] = idx_v[f, g, s16] + f * VOCAB
        return 0

    lax.fori_loop(0, N_FIELDS * _NG, add_off, 0)

    def do_field(f, rows_v, wsem, first):
        @pl.when(jnp.logical_not(first))
        def _():
            pltpu.make_async_copy(
                rows_v, out_hbm.at[pl.ds(base, _BPW), pl.ds(0, EMBED)], wsem
            ).wait()

        copies = [
            pltpu.async_copy(
                tab_hbm.at[idx_v.at[f, g]],
                rows_v.at[pl.ds(g * _GSZ, _GSZ)],
                gsem,
            )
            for g in range(_NG)
        ]
        for c in copies:
            c.wait()
        pltpu.async_copy(
            rows_v, out_hbm.at[pl.ds(base, _BPW), pl.ds(f * EMBED, EMBED)], wsem
        )

    def pair(p, _):
        do_field(2 * p, rows0, wsem0, p == 0)
        do_field(2 * p + 1, rows1, wsem1, p == 0)
        return 0

    lax.fori_loop(0, N_FIELDS // 2, pair, 0)

    pltpu.make_async_copy(
        rows0, out_hbm.at[pl.ds(base, _BPW), pl.ds(0, EMBED)], wsem0
    ).wait()
    pltpu.make_async_copy(
        rows1, out_hbm.at[pl.ds(base, _BPW), pl.ds(0, EMBED)], wsem1
    ).wait()


@functools.partial(jax.jit, static_argnums=())
def kernel(ids, tables):
    ids4 = ids.reshape(N_FIELDS, _NW, _NG, _GSZ)
    tab = tables.reshape(N_FIELDS * VOCAB, EMBED)
    run = pl.kernel(
        _body,
        out_type=jax.ShapeDtypeStruct((BATCH, N_FIELDS * EMBED), jnp.float32),
        mesh=plsc.VectorSubcoreMesh(core_axis_name="c", subcore_axis_name="s"),
        scratch_types=[
            pltpu.VMEM((N_FIELDS, _NG, _GSZ), jnp.int32),
            pltpu.VMEM((_BPW, EMBED), jnp.float32),
            pltpu.VMEM((_BPW, EMBED), jnp.float32),
            pltpu.SemaphoreType.DMA,
            pltpu.SemaphoreType.DMA,
            pltpu.SemaphoreType.DMA,
        ],
        compiler_params=pltpu.CompilerParams(use_tc_tiling_on_sc=False),
    )
    return run(ids4, tab)
